# baseline (speedup 1.0000x reference)
import functools

import jax
import jax.numpy as jnp
from jax import lax
from jax.experimental import pallas as pl
from jax.experimental.pallas import tpu as pltpu
from jax.experimental.pallas import tpu_sc as plsc

_V = 1001
_D = 128
_B = 16384
_NC = 2
_NS = 16
_NW = _NC * _NS

_TC_ROWS = 8192
_SC_ROWS = _B - _TC_ROWS

_BPW = _SC_ROWS // _NW
_CHUNK = 128
_NCH = _BPW // _CHUNK

_TC_BLK = 512
_TC_NBLK = _TC_ROWS // _TC_BLK

_mesh = plsc.VectorSubcoreMesh(core_axis_name="c", subcore_axis_name="s")


@functools.partial(
    pl.kernel,
    mesh=_mesh,
    out_type=jax.ShapeDtypeStruct((_SC_ROWS // _CHUNK, _CHUNK, _D), jnp.float32),
    scratch_types=[
        pltpu.VMEM((_NCH, _CHUNK), jnp.int32),
        pltpu.VMEM((_NCH, _CHUNK, _D), jnp.float32),
        pltpu.SemaphoreType.DMA,
        pltpu.SemaphoreType.DMA,
    ],
)
def _sc_gather(labels_hbm, table_hbm, out_hbm, idx_v, rows_v, gsem, osem):
    wid = lax.axis_index("s") * _NC + lax.axis_index("c")
    pltpu.sync_copy(labels_hbm.at[wid], idx_v)
    gathers = [
        pltpu.async_copy(table_hbm.at[idx_v.at[j]], rows_v.at[j], gsem)
        for j in range(_NCH)
    ]
    outs = []
    for j in range(_NCH):
        gathers[j].wait()
        outs.append(
            pltpu.async_copy(rows_v.at[j], out_hbm.at[wid * _NCH + j], osem)
        )
    for o in outs:
        o.wait()


def _tc_body(lab_ref, tab_ref, out_ref):
    lab = lab_ref[0, 0, :]
    onehot = (
        lax.broadcasted_iota(jnp.int32, (_TC_BLK, _V), 1) == lab[:, None]
    ).astype(jnp.float32)
    out_ref[...] = jnp.dot(
        onehot, tab_ref[...], preferred_element_type=jnp.float32
    )


_tc_gather = pl.pallas_call(
    _tc_body,
    grid=(_TC_NBLK,),
    in_specs=[
        pl.BlockSpec((1, 1, _TC_BLK), lambda i: (i, 0, 0)),
        pl.BlockSpec((_V, _D), lambda i: (0, 0)),
    ],
    out_specs=pl.BlockSpec((_TC_BLK, _D), lambda i: (i, 0)),
    out_shape=jax.ShapeDtypeStruct((_TC_ROWS, _D), jnp.float32),
)


def kernel(labels, train, dtype, table):
    del train
    labels = labels.astype(jnp.int32)
    sc_labels = labels[_TC_ROWS:].reshape(_NW, _NCH, _CHUNK)
    sc_out = _sc_gather(sc_labels, table)
    tc_labels = labels[:_TC_ROWS].reshape(_TC_NBLK, 1, _TC_BLK)
    tc_out = _tc_gather(tc_labels, table)
    out = jnp.concatenate([tc_out, sc_out.reshape(_SC_ROWS, _D)], axis=0)
    return out.astype(dtype.dtype)
